# baseline (speedup 1.0000x reference)
import jax
import jax.numpy as jnp
from jax.experimental import pallas as pl
from jax.experimental.pallas import tpu as pltpu

_NE = 1024
_D = 64
_HW = 1024
_B = 32
_BB = 8
_P = _BB * _HW
_STEPS = _B // _BB
_N = _B * _HW


def _vq_body(x_ref, w_ref,
             qst_ref, idx_ref, loss_ref, perp_ref,
             counts_ref, acc_ref, sw_ref):
    step = pl.program_id(0)
    x = jnp.concatenate([x_ref[i] for i in range(_BB)], axis=1)
    w = w_ref[...]

    @pl.when(step == 0)
    def _init_sw():
        sw_ref[...] = jnp.sum(w * w, axis=1, keepdims=True)

    sw = sw_ref[...]

    sx = jnp.sum(x * x, axis=0, keepdims=True)
    mm = jax.lax.dot_general(w, x, (((1,), (0,)), ((), ())),
                             preferred_element_type=jnp.float32)
    d = (sx + sw) - 2.0 * mm

    m = jnp.min(d, axis=0, keepdims=True)
    iota = jax.lax.broadcasted_iota(jnp.int32, (_NE, _P), 0)
    idx = jnp.min(jnp.where(d == m, iota, _NE), axis=0)
    onehot = (iota == idx[None, :]).astype(jnp.bfloat16)

    q = jax.lax.dot_general(w, onehot, (((0,), (0,)), ((), ())),
                            preferred_element_type=jnp.float32)
    qst = x + (q - x)
    for i in range(_BB):
        qst_ref[i] = qst[:, i * _HW:(i + 1) * _HW].reshape(_D, 32, 32)
        idx_ref[i] = idx[None, i * _HW:(i + 1) * _HW]

    @pl.when(step == 0)
    def _init():
        acc_ref[0, 0] = 0.0
        counts_ref[...] = jnp.zeros_like(counts_ref)

    acc_ref[0, 0] += jnp.sum(m)
    counts_ref[...] += jnp.sum(onehot.astype(jnp.float32), axis=1,
                               keepdims=True)

    @pl.when(step == _STEPS - 1)
    def _finish():
        loss_ref[...] = (acc_ref[0, 0] * (1.25 / (_N * _D))).reshape(1, 1)
        p = counts_ref[...] * (1.0 / _N)
        ent = jnp.sum(p * jnp.log(p + 1e-10), keepdims=True)
        perp_ref[...] = jnp.exp(-ent).reshape(1, 1)


def kernel(inputs, W):
    B, C, H, Wd = inputs.shape
    x3 = inputs.reshape(B, C, H * Wd)

    qst3, idx3, loss, perp = pl.pallas_call(
        _vq_body,
        grid=(_STEPS,),
        in_specs=[
            pl.BlockSpec((_BB, _D, _HW), lambda i: (i, 0, 0)),
            pl.BlockSpec((_NE, _D), lambda i: (0, 0)),
        ],
        out_specs=[
            pl.BlockSpec((_BB, _D, 32, 32), lambda i: (i, 0, 0, 0)),
            pl.BlockSpec((_BB, 1, _HW), lambda i: (i, 0, 0)),
            pl.BlockSpec((1, 1), lambda i: (0, 0)),
            pl.BlockSpec((1, 1), lambda i: (0, 0)),
        ],
        out_shape=[
            jax.ShapeDtypeStruct((_B, _D, 32, 32), jnp.float32),
            jax.ShapeDtypeStruct((_B, 1, _HW), jnp.int32),
            jax.ShapeDtypeStruct((1, 1), jnp.float32),
            jax.ShapeDtypeStruct((1, 1), jnp.float32),
        ],
        scratch_shapes=[
            pltpu.VMEM((_NE, 1), jnp.float32),
            pltpu.SMEM((1, 1), jnp.float32),
            pltpu.VMEM((_NE, 1), jnp.float32),
        ],
    )(x3, W)

    qst = qst3
    idx = idx3.reshape(-1)[:, None]
    return (loss.reshape(()), qst, perp.reshape(()), idx)


# speedup vs baseline: 1.0304x; 1.0304x over previous
import jax
import jax.numpy as jnp
from jax.experimental import pallas as pl
from jax.experimental.pallas import tpu as pltpu

_NE = 1024
_D = 64
_N = 32 * 32 * 32
_P = 4096
_STEPS = _N // _P


def _vq_body(x_ref, w_ref, sw_ref,
             qst_ref, idx_ref, loss_ref, perp_ref,
             counts_ref, acc_ref):
    step = pl.program_id(0)
    x = x_ref[...]
    w = w_ref[...]
    sw = sw_ref[...]

    sx = jnp.sum(x * x, axis=1, keepdims=True)
    mm = jax.lax.dot_general(x, w, (((1,), (1,)), ((), ())),
                             preferred_element_type=jnp.float32)
    d = (sx + sw) - 2.0 * mm

    m = jnp.min(d, axis=1, keepdims=True)
    iota = jax.lax.broadcasted_iota(jnp.int32, (_P, _NE), 1)
    idx = jnp.min(jnp.where(d == m, iota, _NE), axis=1)
    onehot = (iota == idx[:, None]).astype(jnp.bfloat16)

    q = jax.lax.dot_general(onehot, w, (((1,), (0,)), ((), ())),
                            preferred_element_type=jnp.float32)
    qst_ref[...] = x + (q - x)
    idx_ref[...] = idx[:, None]

    @pl.when(step == 0)
    def _init():
        acc_ref[0, 0] = 0.0
        counts_ref[...] = jnp.zeros_like(counts_ref)

    acc_ref[0, 0] += jnp.sum(m)
    counts_ref[...] += jnp.sum(onehot.astype(jnp.float32), axis=0,
                               keepdims=True)

    @pl.when(step == _STEPS - 1)
    def _finish():
        loss_ref[...] = (acc_ref[0, 0] * (1.25 / (_N * _D))).reshape(1, 1)
        p = counts_ref[...] * (1.0 / _N)
        ent = jnp.sum(p * jnp.log(p + 1e-10), keepdims=True)
        perp_ref[...] = jnp.exp(-ent).reshape(1, 1)


def kernel(inputs, W):
    B, C, H, Wd = inputs.shape
    flat = jnp.transpose(inputs, (0, 2, 3, 1)).reshape(-1, C)
    sw = jnp.sum(W ** 2, axis=1)[None, :]

    qst_flat, idx, loss, perp = pl.pallas_call(
        _vq_body,
        grid=(_STEPS,),
        in_specs=[
            pl.BlockSpec((_P, _D), lambda i: (i, 0)),
            pl.BlockSpec((_NE, _D), lambda i: (0, 0)),
            pl.BlockSpec((1, _NE), lambda i: (0, 0)),
        ],
        out_specs=[
            pl.BlockSpec((_P, _D), lambda i: (i, 0)),
            pl.BlockSpec((_P, 1), lambda i: (i, 0)),
            pl.BlockSpec((1, 1), lambda i: (0, 0)),
            pl.BlockSpec((1, 1), lambda i: (0, 0)),
        ],
        out_shape=[
            jax.ShapeDtypeStruct((_N, _D), jnp.float32),
            jax.ShapeDtypeStruct((_N, 1), jnp.int32),
            jax.ShapeDtypeStruct((1, 1), jnp.float32),
            jax.ShapeDtypeStruct((1, 1), jnp.float32),
        ],
        scratch_shapes=[
            pltpu.VMEM((1, _NE), jnp.float32),
            pltpu.SMEM((1, 1), jnp.float32),
        ],
    )(flat, W, sw)

    qst = jnp.transpose(qst_flat.reshape(B, H, Wd, C), (0, 3, 1, 2))
    return (loss.reshape(()), qst, perp.reshape(()), idx)


# speedup vs baseline: 1.3517x; 1.3118x over previous
import jax
import jax.numpy as jnp
from jax.experimental import pallas as pl
from jax.experimental.pallas import tpu as pltpu

_NE = 1024
_D = 64
_N = 32 * 32 * 32
_P = 4096
_STEPS = _N // _P


def _vq_body(x_ref, w_ref,
             qst_ref, idx_ref, loss_ref, perp_ref,
             counts_ref, acc_ref, sw_ref):
    step = pl.program_id(0)
    x = x_ref[...]
    w = w_ref[...]

    @pl.when(step == 0)
    def _init_sw():
        sw_ref[...] = jnp.sum(w * w, axis=1, keepdims=True)

    sw = sw_ref[...]

    ones = jnp.ones((1, _D), dtype=jnp.float32)
    sx = jax.lax.dot_general(ones, x * x, (((1,), (1,)), ((), ())),
                             preferred_element_type=jnp.float32)
    mm = jax.lax.dot_general(w, x, (((1,), (1,)), ((), ())),
                             preferred_element_type=jnp.float32)
    d = (sx + sw) - 2.0 * mm

    m = jnp.min(d, axis=0, keepdims=True)
    iota = jax.lax.broadcasted_iota(jnp.int32, (_NE, _P), 0)
    idx = jnp.min(jnp.where(d == m, iota, _NE), axis=0)
    onehot = (iota == idx[None, :]).astype(jnp.bfloat16)

    q = jax.lax.dot_general(onehot, w, (((0,), (0,)), ((), ())),
                            preferred_element_type=jnp.float32)
    qst_ref[...] = x + (q - x)
    idx_ref[0] = idx[None, :]

    @pl.when(step == 0)
    def _init():
        acc_ref[0, 0] = 0.0
        counts_ref[...] = jnp.zeros_like(counts_ref)

    acc_ref[0, 0] += jnp.sum(m)
    counts_ref[...] += jnp.sum(onehot.astype(jnp.float32), axis=1,
                               keepdims=True)

    @pl.when(step == _STEPS - 1)
    def _finish():
        loss_ref[...] = (acc_ref[0, 0] * (1.25 / (_N * _D))).reshape(1, 1)
        p = counts_ref[...] * (1.0 / _N)
        ent = jnp.sum(p * jnp.log(p + 1e-10), keepdims=True)
        perp_ref[...] = jnp.exp(-ent).reshape(1, 1)


def kernel(inputs, W):
    B, C, H, Wd = inputs.shape
    flat = jnp.transpose(inputs, (0, 2, 3, 1)).reshape(-1, C)

    qst_flat, idx3, loss, perp = pl.pallas_call(
        _vq_body,
        grid=(_STEPS,),
        in_specs=[
            pl.BlockSpec((_P, _D), lambda i: (i, 0)),
            pl.BlockSpec((_NE, _D), lambda i: (0, 0)),
        ],
        out_specs=[
            pl.BlockSpec((_P, _D), lambda i: (i, 0)),
            pl.BlockSpec((1, 1, _P), lambda i: (i, 0, 0)),
            pl.BlockSpec((1, 1), lambda i: (0, 0)),
            pl.BlockSpec((1, 1), lambda i: (0, 0)),
        ],
        out_shape=[
            jax.ShapeDtypeStruct((_N, _D), jnp.float32),
            jax.ShapeDtypeStruct((_STEPS, 1, _P), jnp.int32),
            jax.ShapeDtypeStruct((1, 1), jnp.float32),
            jax.ShapeDtypeStruct((1, 1), jnp.float32),
        ],
        scratch_shapes=[
            pltpu.VMEM((_NE, 1), jnp.float32),
            pltpu.SMEM((1, 1), jnp.float32),
            pltpu.VMEM((_NE, 1), jnp.float32),
        ],
    )(flat, W)

    qst = jnp.transpose(qst_flat.reshape(B, H, Wd, C), (0, 3, 1, 2))
    idx = idx3.reshape(-1)[:, None]
    return (loss.reshape(()), qst, perp.reshape(()), idx)
